# baseline (speedup 1.0000x reference)
import jax
import jax.numpy as jnp
from jax.experimental import pallas as pl
from jax.experimental.pallas import tpu as pltpu

B = 8
N = 16384
C = 3
NPTS = 1024
BIG = 1e10


def _argmax_tuple(d, xs, ys, zs):
    w = d.shape[1]
    while w > 1:
        h = w // 2
        dl = d[:, :h]
        dr = d[:, h:]
        take_r = dr > dl
        d = jnp.where(take_r, dr, dl)
        xs = jnp.where(take_r, xs[:, h:], xs[:, :h])
        ys = jnp.where(take_r, ys[:, h:], ys[:, :h])
        zs = jnp.where(take_r, zs[:, h:], zs[:, :h])
        w = h
    return xs, ys, zs


def _fps_body(x_ref, out_ref):
    X = x_ref[0]
    Y = x_ref[1]
    Z = x_ref[2]

    iota_p = jax.lax.broadcasted_iota(jnp.int32, (B, NPTS), 1)

    qx0 = X[:, 0:1]
    qy0 = Y[:, 0:1]
    qz0 = Z[:, 0:1]
    zeros_p = jnp.zeros((B, NPTS), dtype=jnp.float32)
    ox0 = jnp.where(iota_p == 0, qx0, zeros_p)
    oy0 = jnp.where(iota_p == 0, qy0, zeros_p)
    oz0 = jnp.where(iota_p == 0, qz0, zeros_p)
    dists0 = jnp.full((B, N), BIG, dtype=jnp.float32)

    def body(i, carry):
        dists, ox, oy, oz, qx, qy, qz = carry
        dx = X - qx
        dy = Y - qy
        dz = Z - qz
        d = dx * dx + dy * dy + dz * dz
        dists = jnp.minimum(dists, d)
        qx, qy, qz = _argmax_tuple(dists, X, Y, Z)
        osel = iota_p == (i + 1)
        ox = jnp.where(osel, qx, ox)
        oy = jnp.where(osel, qy, oy)
        oz = jnp.where(osel, qz, oz)
        return dists, ox, oy, oz, qx, qy, qz

    carry = (dists0, ox0, oy0, oz0, qx0, qy0, qz0)
    _, ox, oy, oz, _, _, _ = jax.lax.fori_loop(0, NPTS - 1, body, carry)
    out_ref[0] = ox
    out_ref[1] = oy
    out_ref[2] = oz


def kernel(x):
    xt = jnp.transpose(x, (1, 0, 2))
    out = pl.pallas_call(
        _fps_body,
        out_shape=jax.ShapeDtypeStruct((C, B, NPTS), jnp.float32),
    )(xt)
    return jnp.transpose(out, (1, 0, 2))


# speedup vs baseline: 2.1069x; 2.1069x over previous
import jax
import jax.numpy as jnp
from jax.experimental import pallas as pl
from jax.experimental.pallas import tpu as pltpu

B = 8
N = 16384
C = 3
NPTS = 1024
BIG = 1e10
CH = 512
NCH = N // CH
NACC = 2


def _merge(a, b):
    av, ai, ax, ay, az = a
    bv, bi, bx, by, bz = b
    take_b = (bv > av) | ((bv == av) & (bi < ai))
    return (
        jnp.where(take_b, bv, av),
        jnp.where(take_b, bi, ai),
        jnp.where(take_b, bx, ax),
        jnp.where(take_b, by, ay),
        jnp.where(take_b, bz, az),
    )


def _fps_body(x_ref, out_ref, dists_ref):
    iota_p = jax.lax.broadcasted_iota(jnp.int32, (B, NPTS), 1)
    iota_c = jax.lax.broadcasted_iota(jnp.int32, (B, CH), 1)

    qx0 = x_ref[0, :, 0:1]
    qy0 = x_ref[1, :, 0:1]
    qz0 = x_ref[2, :, 0:1]
    zeros_p = jnp.zeros((B, NPTS), dtype=jnp.float32)
    out_ref[0] = jnp.where(iota_p == 0, qx0, zeros_p)
    out_ref[1] = jnp.where(iota_p == 0, qy0, zeros_p)
    out_ref[2] = jnp.where(iota_p == 0, qz0, zeros_p)
    dists_ref[...] = jnp.full((B, N), BIG, dtype=jnp.float32)

    def body(i, q):
        qx, qy, qz = q
        accs = [None] * NACC
        for c in range(NCH):
            sl = slice(c * CH, (c + 1) * CH)
            xc = x_ref[0, :, sl]
            yc = x_ref[1, :, sl]
            zc = x_ref[2, :, sl]
            dx = xc - qx
            dy = yc - qy
            dz = zc - qz
            d = dx * dx + dy * dy + dz * dz
            dn = jnp.minimum(dists_ref[:, sl], d)
            dists_ref[:, sl] = dn
            gi = iota_c + (c * CH)
            k = c % NACC
            if accs[k] is None:
                accs[k] = (dn, gi, xc, yc, zc)
            else:
                av, ai, ax, ay, az = accs[k]
                t = dn > av
                accs[k] = (
                    jnp.where(t, dn, av),
                    jnp.where(t, gi, ai),
                    jnp.where(t, xc, ax),
                    jnp.where(t, yc, ay),
                    jnp.where(t, zc, az),
                )
        acc = accs[0]
        for k in range(1, NACC):
            acc = _merge(acc, accs[k])
        w = CH
        while w > 128:
            h = w // 2
            acc = _merge(tuple(t[:, :h] for t in acc),
                         tuple(t[:, h:] for t in acc))
            w = h
        av, ai, ax, ay, az = acc
        def _lex(a, b):
            (a_v, a_i), (b_v, b_i) = a, b
            tb = (b_v > a_v) | ((b_v == a_v) & (b_i < a_i))
            return jnp.where(tb, b_v, a_v), jnp.where(tb, b_i, a_i)

        def _stage(pair, shifts):
            cands = [pair] + [
                (pltpu.roll(pair[0], s, 1), pltpu.roll(pair[1], s, 1))
                for s in shifts
            ]
            while len(cands) > 1:
                cands = [_lex(cands[j], cands[j + 1])
                         for j in range(0, len(cands) - 1, 2)] + (
                             [cands[-1]] if len(cands) % 2 else [])
            return cands[0]

        p1 = _stage((av, ai), [8 * k for k in range(1, 16)])
        _, nxt = _stage(p1, list(range(1, 8)))
        pos = jnp.bitwise_and(nxt, 127)
        qx = jnp.take_along_axis(ax, pos, axis=1)
        qy = jnp.take_along_axis(ay, pos, axis=1)
        qz = jnp.take_along_axis(az, pos, axis=1)
        osel = iota_p == (i + 1)
        qxp = jnp.concatenate([qx] * (NPTS // 128), axis=1)
        qyp = jnp.concatenate([qy] * (NPTS // 128), axis=1)
        qzp = jnp.concatenate([qz] * (NPTS // 128), axis=1)
        out_ref[0] = jnp.where(osel, qxp, out_ref[0])
        out_ref[1] = jnp.where(osel, qyp, out_ref[1])
        out_ref[2] = jnp.where(osel, qzp, out_ref[2])
        qxc = jnp.concatenate([qx] * (CH // 128), axis=1)
        qyc = jnp.concatenate([qy] * (CH // 128), axis=1)
        qzc = jnp.concatenate([qz] * (CH // 128), axis=1)
        return qxc, qyc, qzc

    q0 = (jnp.broadcast_to(qx0, (B, CH)),
          jnp.broadcast_to(qy0, (B, CH)),
          jnp.broadcast_to(qz0, (B, CH)))
    jax.lax.fori_loop(0, NPTS - 1, body, q0)


def kernel(x):
    xt = jnp.transpose(x, (1, 0, 2))
    out = pl.pallas_call(
        _fps_body,
        out_shape=jax.ShapeDtypeStruct((C, B, NPTS), jnp.float32),
        scratch_shapes=[pltpu.VMEM((B, N), jnp.float32)],
    )(xt)
    return jnp.transpose(out, (1, 0, 2))
